# baseline (speedup 1.0000x reference)
import functools

import jax
import jax.numpy as jnp
from jax.experimental import pallas as pl

_NUM_EXPERT = 64
_BLOCK = 2048


def _router_body(x_ref, w_ref, b_ref, i1_ref, i2_ref, s1_ref, s2_ref):
    x = x_ref[...]
    w = w_ref[...]
    b = b_ref[...]
    logits = jax.lax.dot_general(
        x, w, (((1,), (1,)), ((), ())),
        preferred_element_type=jnp.float32) + b

    lt = logits.T
    sub = jax.lax.broadcasted_iota(jnp.int32, lt.shape, 0)
    v1 = jnp.max(lt, axis=0, keepdims=True)
    i1 = jnp.argmax(lt, axis=0).reshape(1, -1)
    masked = jnp.where(sub == i1, -jnp.inf, lt)
    v2 = jnp.max(masked, axis=0, keepdims=True)
    i2 = jnp.argmax(masked, axis=0).reshape(1, -1)

    denom = jnp.sum(jnp.exp(lt - v1), axis=0, keepdims=True)
    s1 = 1.0 / denom
    s2 = jnp.exp(v2 - v1) / denom

    i1_ref[...] = i1.reshape(1, 1, _BLOCK)
    i2_ref[...] = i2.reshape(1, 1, _BLOCK)
    s1_ref[...] = s1.reshape(1, 1, _BLOCK)
    s2_ref[...] = s2.reshape(1, 1, _BLOCK)


@jax.jit
def _run(inp, W, b2d):
    n_tokens, d_model = inp.shape
    nblk = n_tokens // _BLOCK
    plane_i = jax.ShapeDtypeStruct((nblk, 1, _BLOCK), jnp.int32)
    plane_f = jax.ShapeDtypeStruct((nblk, 1, _BLOCK), jnp.float32)
    out_spec = pl.BlockSpec((1, 1, _BLOCK), lambda i: (i, 0, 0))
    return pl.pallas_call(
        _router_body,
        grid=(nblk,),
        in_specs=[
            pl.BlockSpec((_BLOCK, d_model), lambda i: (i, 0)),
            pl.BlockSpec((_NUM_EXPERT, d_model), lambda i: (0, 0)),
            pl.BlockSpec((1, _NUM_EXPERT), lambda i: (0, 0)),
        ],
        out_specs=[out_spec, out_spec, out_spec, out_spec],
        out_shape=[plane_i, plane_i, plane_f, plane_f],
    )(inp, W, b2d)


def kernel(inp, W, b):
    i1, i2, s1, s2 = _run(inp, W, b.reshape(1, -1))
    n = inp.shape[0]
    idx = jnp.stack([i1.reshape(n), i2.reshape(n)], axis=-1)
    val = jnp.stack([s1.reshape(n), s2.reshape(n)], axis=-1)
    return idx, val


# speedup vs baseline: 1.0411x; 1.0411x over previous
import functools

import jax
import jax.numpy as jnp
from jax.experimental import pallas as pl

_NUM_EXPERT = 64
_BLOCK = 4096


def _router_body(x_ref, w_ref, b_ref, i1_ref, i2_ref, s1_ref, s2_ref):
    x = x_ref[...]
    w = w_ref[...]
    b = b_ref[...]
    logits = jax.lax.dot_general(
        x, w, (((1,), (1,)), ((), ())),
        preferred_element_type=jnp.float32) + b

    lt = logits.T
    sub = jax.lax.broadcasted_iota(jnp.int32, lt.shape, 0)
    v1 = jnp.max(lt, axis=0, keepdims=True)
    i1 = jnp.argmax(lt, axis=0).reshape(1, -1)
    masked = jnp.where(sub == i1, -jnp.inf, lt)
    v2 = jnp.max(masked, axis=0, keepdims=True)
    i2 = jnp.argmax(masked, axis=0).reshape(1, -1)

    denom = jnp.sum(jnp.exp(lt - v1), axis=0, keepdims=True)
    s1 = 1.0 / denom
    s2 = jnp.exp(v2 - v1) / denom

    i1_ref[...] = i1.reshape(1, 1, _BLOCK)
    i2_ref[...] = i2.reshape(1, 1, _BLOCK)
    s1_ref[...] = s1.reshape(1, 1, _BLOCK)
    s2_ref[...] = s2.reshape(1, 1, _BLOCK)


@jax.jit
def _run(inp, W, b2d):
    n_tokens, d_model = inp.shape
    nblk = n_tokens // _BLOCK
    plane_i = jax.ShapeDtypeStruct((nblk, 1, _BLOCK), jnp.int32)
    plane_f = jax.ShapeDtypeStruct((nblk, 1, _BLOCK), jnp.float32)
    out_spec = pl.BlockSpec((1, 1, _BLOCK), lambda i: (i, 0, 0))
    return pl.pallas_call(
        _router_body,
        grid=(nblk,),
        in_specs=[
            pl.BlockSpec((_BLOCK, d_model), lambda i: (i, 0)),
            pl.BlockSpec((_NUM_EXPERT, d_model), lambda i: (0, 0)),
            pl.BlockSpec((1, _NUM_EXPERT), lambda i: (0, 0)),
        ],
        out_specs=[out_spec, out_spec, out_spec, out_spec],
        out_shape=[plane_i, plane_i, plane_f, plane_f],
    )(inp, W, b2d)


def kernel(inp, W, b):
    i1, i2, s1, s2 = _run(inp, W, b.reshape(1, -1))
    n = inp.shape[0]
    idx = jnp.stack([i1.reshape(n), i2.reshape(n)], axis=-1)
    val = jnp.stack([s1.reshape(n), s2.reshape(n)], axis=-1)
    return idx, val


# speedup vs baseline: 1.0447x; 1.0034x over previous
import functools

import jax
import jax.numpy as jnp
from jax.experimental import pallas as pl

_NUM_EXPERT = 64
_BLOCK = 4096


def _router_body(x_ref, w_ref, b_ref, idx_ref, val_ref):
    x = x_ref[...]
    w = w_ref[...]
    b = b_ref[...]
    logits = jax.lax.dot_general(
        x, w, (((1,), (1,)), ((), ())),
        preferred_element_type=jnp.float32) + b

    lt = logits.T
    sub = jax.lax.broadcasted_iota(jnp.int32, lt.shape, 0)
    v1 = jnp.max(lt, axis=0, keepdims=True)
    i1 = jnp.argmax(lt, axis=0).reshape(1, -1)
    masked = jnp.where(sub == i1, -jnp.inf, lt)
    v2 = jnp.max(masked, axis=0, keepdims=True)
    i2 = jnp.argmax(masked, axis=0).reshape(1, -1)

    denom = jnp.sum(jnp.exp(lt - v1), axis=0, keepdims=True)
    s1 = 1.0 / denom
    s2 = jnp.exp(v2 - v1) / denom

    idx_ref[...] = jnp.concatenate([i1, i2], axis=1).reshape(1, 1, 2 * _BLOCK)
    val_ref[...] = jnp.concatenate([s1, s2], axis=1).reshape(1, 1, 2 * _BLOCK)


@jax.jit
def _run(inp, W, b2d):
    n_tokens, d_model = inp.shape
    nblk = n_tokens // _BLOCK
    plane_i = jax.ShapeDtypeStruct((nblk, 1, 2 * _BLOCK), jnp.int32)
    plane_f = jax.ShapeDtypeStruct((nblk, 1, 2 * _BLOCK), jnp.float32)
    out_spec = pl.BlockSpec((1, 1, 2 * _BLOCK), lambda i: (i, 0, 0))
    return pl.pallas_call(
        _router_body,
        grid=(nblk,),
        in_specs=[
            pl.BlockSpec((_BLOCK, d_model), lambda i: (i, 0)),
            pl.BlockSpec((_NUM_EXPERT, d_model), lambda i: (0, 0)),
            pl.BlockSpec((1, _NUM_EXPERT), lambda i: (0, 0)),
        ],
        out_specs=[out_spec, out_spec],
        out_shape=[plane_i, plane_f],
    )(inp, W, b2d)


def kernel(inp, W, b):
    idxp, valp = _run(inp, W, b.reshape(1, -1))
    n = inp.shape[0]
    idx = jnp.stack([idxp[:, 0, :_BLOCK].reshape(n),
                     idxp[:, 0, _BLOCK:].reshape(n)], axis=-1)
    val = jnp.stack([valp[:, 0, :_BLOCK].reshape(n),
                     valp[:, 0, _BLOCK:].reshape(n)], axis=-1)
    return idx, val
